# baseline (speedup 1.0000x reference)
import functools

import jax
import jax.numpy as jnp
from jax import lax
from jax.experimental import pallas as pl
from jax.experimental.pallas import tpu as pltpu
from jax.experimental.pallas import tpu_sc as plsc

_N = 5000
_B = 512
_NB = 10
_NT = _B * _NB
_TH = 0.7
_CW = 2560
_D = 16
_NW = 32
_BPW = _NT // _NW


@functools.partial(
    pl.kernel,
    mesh=plsc.VectorSubcoreMesh(core_axis_name="c", subcore_axis_name="s"),
    compiler_params=pltpu.CompilerParams(use_tc_tiling_on_sc=False),
    out_type=jax.ShapeDtypeStruct((_NT, _D), jnp.float32),
    scratch_types=[
        pltpu.VMEM((_BPW,), jnp.int32),
        pltpu.VMEM((_BPW, _D), jnp.float32),
        pltpu.SemaphoreType.DMA,
    ],
)
def _sc_gather(table_hbm, idx_hbm, out_hbm, idx_v, rows_v, sem):
    wid = lax.axis_index("s") * 2 + lax.axis_index("c")
    base = wid * _BPW
    pltpu.sync_copy(idx_hbm.at[pl.ds(base, _BPW)], idx_v)
    pltpu.async_copy(table_hbm.at[idx_v], rows_v, sem).wait()
    pltpu.sync_copy(rows_v, out_hbm.at[pl.ds(base, _BPW)])


def _overlap(px1, py1, px2, py2, pa, rows_ref, off, w):
    x1r = rows_ref[0:1, off:off + w]
    y1r = rows_ref[1:2, off:off + w]
    x2r = rows_ref[2:3, off:off + w]
    y2r = rows_ref[3:4, off:off + w]
    ar = rows_ref[5:6, off:off + w]
    ix1 = jnp.maximum(px1, x1r)
    iy1 = jnp.maximum(py1, y1r)
    ix2 = jnp.minimum(px2, x2r)
    iy2 = jnp.minimum(py2, y2r)
    inter = jnp.maximum(ix2 - ix1, 0.0) * jnp.maximum(iy2 - iy1, 0.0)
    union = pa + ar - inter
    return ((inter / union) > _TH).astype(jnp.float32)


def _nms_body(rows_ref, cols_ref, keep_ref):
    rid = lax.broadcasted_iota(jnp.int32, (_B, _B), 0)
    cid = lax.broadcasted_iota(jnp.int32, (_B, _B), 1)
    tri = (rid < cid).astype(jnp.float32)
    lcol = lax.broadcasted_iota(jnp.int32, (1, _CW), 1)

    def nms_pass(exact):
        keep_ref[...] = jnp.ones((1, _NT), jnp.float32)
        acc = jnp.zeros((1, _B), jnp.float32)

        for i in range(_NB):
            base = i * _B
            c = cols_ref[i, :, :]
            px1 = c[:, 0:1]
            py1 = c[:, 1:2]
            px2 = c[:, 2:3]
            py2 = c[:, 3:4]
            pa = c[:, 5:6]

            rest = _NT - base
            widths = []
            while rest > 0:
                widths.append(min(_CW, rest))
                rest -= widths[-1]

            ov0 = _overlap(px1, py1, px2, py2, pa, rows_ref, base, widths[0])
            om = ov0[:, 0:_B] * tri
            pre = keep_ref[0:1, base:base + _B]

            def fix(kv, om=om, pre=pre):
                s = lax.dot_general(kv, om, (((1,), (0,)), ((), ())),
                                    preferred_element_type=jnp.float32)
                return pre * (s == 0.0).astype(jnp.float32)

            if exact:
                kv_a = fix(pre)
                kv = fix(kv_a)

                def fix_cond(carry):
                    return carry[1]

                def fix_body(carry, fix=fix):
                    nk = fix(carry[0])
                    return (nk, jnp.any(nk != carry[0]))

                kv_f, _ = lax.while_loop(fix_cond, fix_body,
                                         (kv, jnp.any(kv != kv_a)))
            else:
                kv = fix(fix(pre))
                kv_f = fix(kv)
                acc = acc + jnp.abs(kv_f - kv)
            keep_ref[0:1, base:base + _B] = kv_f

            off = base
            for ci, w in enumerate(widths):
                ov = ov0 if ci == 0 else _overlap(px1, py1, px2, py2, pa,
                                                  rows_ref, off, w)
                s_all = lax.dot_general(kv_f, ov, (((1,), (0,)), ((), ())),
                                        preferred_element_type=jnp.float32)
                sup = s_all > 0.0
                if ci == 0:
                    sup = sup & (lcol[:, 0:w] >= _B)
                keep_ref[0:1, off:off + w] = (
                    keep_ref[0:1, off:off + w] * (1.0 - sup.astype(jnp.float32)))
                off += w
        return acc

    acc = nms_pass(exact=False)

    @pl.when(jnp.any(acc > 0.0))
    def _redo():
        nms_pass(exact=True)


def _nms_keep(rows, cols):
    return pl.pallas_call(
        _nms_body,
        out_shape=jax.ShapeDtypeStruct((1, _NT), jnp.float32),
    )(rows, cols)


def kernel(boxes, scores):
    order = jnp.argsort(-scores).astype(jnp.int32)
    area = (boxes[:, 2] - boxes[:, 0]) * (boxes[:, 3] - boxes[:, 1])
    table = jnp.concatenate(
        [boxes, scores[:, None], area[:, None],
         jnp.zeros((_N, _D - 6), boxes.dtype)],
        axis=1)
    table = jnp.pad(table, ((0, 8), (0, 0)))
    idx = jnp.concatenate(
        [order, jnp.full((_NT - _N,), _N, jnp.int32)])

    gp = _sc_gather(table, idx)
    cols = gp.reshape(_NB, _B, _D)
    rows = gp.T

    keep = _nms_keep(rows, cols)
    out = gp[:_N, 0:5] * keep[0, :_N, None]
    return out


# speedup vs baseline: 1.0008x; 1.0008x over previous
import functools

import jax
import jax.numpy as jnp
from jax import lax
from jax.experimental import pallas as pl
from jax.experimental.pallas import tpu as pltpu
from jax.experimental.pallas import tpu_sc as plsc

_N = 5000
_B = 512
_NB = 10
_NT = _B * _NB
_TH = 0.7
_CW = 2560
_D = 16
_NW = 32
_BPW = _NT // _NW


@functools.partial(
    pl.kernel,
    mesh=plsc.VectorSubcoreMesh(core_axis_name="c", subcore_axis_name="s"),
    compiler_params=pltpu.CompilerParams(use_tc_tiling_on_sc=False),
    out_type=jax.ShapeDtypeStruct((_NT, _D), jnp.float32),
    scratch_types=[
        pltpu.VMEM((_BPW,), jnp.int32),
        pltpu.VMEM((_BPW, _D), jnp.float32),
        pltpu.SemaphoreType.DMA,
    ],
)
def _sc_gather(table_hbm, idx_hbm, out_hbm, idx_v, rows_v, sem):
    wid = lax.axis_index("s") * 2 + lax.axis_index("c")
    base = wid * _BPW
    pltpu.sync_copy(idx_hbm.at[pl.ds(base, _BPW)], idx_v)
    pltpu.async_copy(table_hbm.at[idx_v], rows_v, sem).wait()
    pltpu.sync_copy(rows_v, out_hbm.at[pl.ds(base, _BPW)])


def _overlap(px1, py1, px2, py2, pa, rows_ref, off, w):
    x1r = rows_ref[0:1, off:off + w]
    y1r = rows_ref[1:2, off:off + w]
    x2r = rows_ref[2:3, off:off + w]
    y2r = rows_ref[3:4, off:off + w]
    ar = rows_ref[5:6, off:off + w]
    ix1 = jnp.maximum(px1, x1r)
    iy1 = jnp.maximum(py1, y1r)
    ix2 = jnp.minimum(px2, x2r)
    iy2 = jnp.minimum(py2, y2r)
    inter = jnp.maximum(ix2 - ix1, 0.0) * jnp.maximum(iy2 - iy1, 0.0)
    union = pa + ar - inter
    return ((inter / union) > _TH).astype(jnp.float32)


def _nms_body(rows_ref, cols_ref, keep_ref):
    rid = lax.broadcasted_iota(jnp.int32, (_B, _B), 0)
    cid = lax.broadcasted_iota(jnp.int32, (_B, _B), 1)
    tri = (rid < cid).astype(jnp.float32)
    lcol = lax.broadcasted_iota(jnp.int32, (1, _CW), 1)

    def nms_pass(exact):
        keep_ref[...] = jnp.ones((1, _NT), jnp.float32)
        acc = jnp.zeros((1, _B), jnp.float32)

        for i in range(_NB):
            base = i * _B
            c = cols_ref[i, :, :]
            px1 = c[:, 0:1]
            py1 = c[:, 1:2]
            px2 = c[:, 2:3]
            py2 = c[:, 3:4]
            pa = c[:, 5:6]

            rest = _NT - base
            widths = []
            while rest > 0:
                widths.append(min(_CW, rest))
                rest -= widths[-1]

            ov0 = _overlap(px1, py1, px2, py2, pa, rows_ref, base, widths[0])
            om = ov0[:, 0:_B] * tri
            pre = keep_ref[0:1, base:base + _B]

            def fix(kv, om=om, pre=pre):
                s = lax.dot_general(kv, om, (((1,), (0,)), ((), ())),
                                    preferred_element_type=jnp.float32)
                return pre * (s == 0.0).astype(jnp.float32)

            if exact:
                kv_a = fix(pre)
                kv = fix(kv_a)

                def fix_cond(carry):
                    return carry[1]

                def fix_body(carry, fix=fix):
                    nk = fix(carry[0])
                    return (nk, jnp.any(nk != carry[0]))

                kv_f, _ = lax.while_loop(fix_cond, fix_body,
                                         (kv, jnp.any(kv != kv_a)))
            else:
                kv = fix(fix(pre))
                kv_f = fix(kv)
                acc = acc + jnp.abs(kv_f - kv)
            keep_ref[0:1, base:base + _B] = kv_f

            off = base
            for ci, w in enumerate(widths):
                ov = ov0 if ci == 0 else _overlap(px1, py1, px2, py2, pa,
                                                  rows_ref, off, w)
                s_all = lax.dot_general(kv_f, ov, (((1,), (0,)), ((), ())),
                                        preferred_element_type=jnp.float32)
                sup = s_all > 0.0
                if ci == 0:
                    sup = sup & (lcol[:, 0:w] >= _B)
                keep_ref[0:1, off:off + w] = (
                    keep_ref[0:1, off:off + w] * (1.0 - sup.astype(jnp.float32)))
                off += w
        return acc

    acc = nms_pass(exact=False)

    @pl.when(jnp.any(acc > 0.0))
    def _redo():
        nms_pass(exact=True)


def _nms_keep(rows, cols):
    return pl.pallas_call(
        _nms_body,
        out_shape=jax.ShapeDtypeStruct((1, _NT), jnp.float32),
    )(rows, cols)


def kernel(boxes, scores):
    order = jnp.argsort(-scores).astype(jnp.int32)
    area = (boxes[:, 2] - boxes[:, 0]) * (boxes[:, 3] - boxes[:, 1])
    table = jnp.concatenate(
        [boxes, scores[:, None], area[:, None],
         jnp.zeros((_N, _D - 6), boxes.dtype)],
        axis=1)
    table = jnp.pad(table, ((0, 8), (0, 0)))
    idx = jnp.concatenate(
        [order, jnp.full((_NT - _N,), _N, jnp.int32)])

    gp = _sc_gather(table, idx)
    cols = gp.reshape(_NB, _B, _D)
    rows = gp[:, 0:8].T

    keep = _nms_keep(rows, cols)
    out = gp[:_N, 0:5] * keep[0, :_N, None]
    return out
